# baseline (speedup 1.0000x reference)
import functools
import math

import jax
import jax.numpy as jnp
from jax.experimental import pallas as pl
from jax.experimental.pallas import tpu as pltpu

_BLOCK_ROWS = 512


def _fused_kernel(x_ref, gb_ref, w_ref, o_ref, *, eps: float):
    x = x_ref[...]
    f = x.shape[-1]
    g = gb_ref[0:1, :]

    s1 = jnp.sum(x, axis=-1, keepdims=True)
    s2 = jnp.sum(x * x, axis=-1, keepdims=True)
    mean = s1 * jnp.float32(1.0 / f)
    var = (s2 - mean * s1) * jnp.float32(1.0 / (f - 1))
    inv = pl.reciprocal(jnp.sqrt(var) + jnp.float32(eps), approx=False)

    xg = (x * g).astype(jnp.bfloat16)
    y = jnp.dot(xg, w_ref[...], preferred_element_type=jnp.float32)

    gbw = jnp.dot(gb_ref[...].astype(jnp.bfloat16), w_ref[...],
                  preferred_element_type=jnp.float32)
    gw = gbw[0:1, :]
    bw = gbw[1:2, :]

    o_ref[...] = x + inv * y + (jnp.float32(-1.0) * inv * mean) * gw + bw


def kernel(x, a_2, b_2, w, eps: float = 1e-6):
    orig_shape = x.shape
    features = orig_shape[-1]
    rows = math.prod(orig_shape[:-1])
    x2 = x.reshape(rows, features)
    gb = jnp.stack([a_2.astype(jnp.float32), b_2.astype(jnp.float32)])
    w_bf16 = w.astype(jnp.bfloat16)

    block_rows = min(_BLOCK_ROWS, rows)
    grid = (pl.cdiv(rows, block_rows),)
    row_spec = pl.BlockSpec((block_rows, features), lambda i: (i, 0))

    out = pl.pallas_call(
        functools.partial(_fused_kernel, eps=eps),
        out_shape=jax.ShapeDtypeStruct((rows, features), x.dtype),
        grid=grid,
        in_specs=[
            row_spec,
            pl.BlockSpec((2, features), lambda i: (0, 0)),
            pl.BlockSpec((features, features), lambda i: (0, 0)),
        ],
        out_specs=row_spec,
        compiler_params=pltpu.CompilerParams(
            dimension_semantics=("arbitrary",),
            vmem_limit_bytes=48 * 1024 * 1024,
        ),
    )(x2, gb, w_bf16)

    return out.reshape(orig_shape)


# speedup vs baseline: 1.0035x; 1.0035x over previous
import functools
import math

import jax
import jax.numpy as jnp
from jax.experimental import pallas as pl
from jax.experimental.pallas import tpu as pltpu

_BLOCK_ROWS = 512


def _fused_kernel(x_ref, gb_ref, w_ref, o_ref, *, eps: float):
    x = x_ref[...]
    f = x.shape[-1]
    g = gb_ref[0:1, :]

    s1 = jnp.sum(x, axis=-1, keepdims=True)
    s2 = jnp.sum(x * x, axis=-1, keepdims=True)
    mean = s1 * jnp.float32(1.0 / f)
    var = (s2 - mean * s1) * jnp.float32(1.0 / (f - 1))
    inv = pl.reciprocal(jnp.sqrt(var) + jnp.float32(eps), approx=False)

    xg = (x * g).astype(jnp.bfloat16)
    y = jnp.dot(xg, w_ref[...], preferred_element_type=jnp.float32)

    gbw = jnp.dot(gb_ref[...].astype(jnp.bfloat16), w_ref[...],
                  preferred_element_type=jnp.float32)
    gw = gbw[0:1, :]
    bw = gbw[1:2, :]

    o_ref[...] = x + inv * y + (jnp.float32(-1.0) * inv * mean) * gw + bw


def kernel(x, a_2, b_2, w, eps: float = 1e-6):
    orig_shape = x.shape
    features = orig_shape[-1]
    rows = math.prod(orig_shape[:-1])
    x2 = x.reshape(rows, features)
    gb = jnp.stack([a_2.astype(jnp.float32), b_2.astype(jnp.float32)])
    w_bf16 = w.astype(jnp.bfloat16)

    block_rows = min(_BLOCK_ROWS, rows)
    nblk = pl.cdiv(rows, block_rows)
    half = max(nblk // 2, 1)
    grid = (pl.cdiv(nblk, half), half)
    row_spec = pl.BlockSpec((block_rows, features), lambda c, i: (c * half + i, 0))

    out = pl.pallas_call(
        functools.partial(_fused_kernel, eps=eps),
        out_shape=jax.ShapeDtypeStruct((rows, features), x.dtype),
        grid=grid,
        in_specs=[
            row_spec,
            pl.BlockSpec((2, features), lambda c, i: (0, 0)),
            pl.BlockSpec((features, features), lambda c, i: (0, 0)),
        ],
        out_specs=row_spec,
        compiler_params=pltpu.CompilerParams(
            dimension_semantics=("parallel", "arbitrary"),
            vmem_limit_bytes=48 * 1024 * 1024,
        ),
    )(x2, gb, w_bf16)

    return out.reshape(orig_shape)


# speedup vs baseline: 1.2313x; 1.2269x over previous
import functools
import math

import jax
import jax.numpy as jnp
from jax.experimental import pallas as pl
from jax.experimental.pallas import tpu as pltpu

_BLOCK_ROWS = 1024


def _fused_ln_ff_residual_kernel(x_ref, g_ref, b_ref, w_ref, o_ref, *, eps: float):
    x = x_ref[...]
    f = x.shape[-1]
    mean = jnp.sum(x, axis=-1, keepdims=True) * jnp.float32(1.0 / f)
    xc = x - mean
    var = jnp.sum(xc * xc, axis=-1, keepdims=True) * jnp.float32(1.0 / (f - 1))
    inv = pl.reciprocal(jnp.sqrt(var) + jnp.float32(eps), approx=False)
    h = xc * inv * g_ref[...] + b_ref[...]
    y = jnp.dot(h.astype(jnp.bfloat16), w_ref[...],
                preferred_element_type=jnp.float32)
    o_ref[...] = x + y


def kernel(x, a_2, b_2, w, eps: float = 1e-6):
    orig_shape = x.shape
    features = orig_shape[-1]
    rows = math.prod(orig_shape[:-1])
    x2 = x.reshape(rows, features)
    g2 = a_2.astype(jnp.float32).reshape(1, features)
    b2 = b_2.astype(jnp.float32).reshape(1, features)
    w_bf16 = w.astype(jnp.bfloat16)

    block_rows = min(_BLOCK_ROWS, rows)
    grid = (pl.cdiv(rows, block_rows),)
    row_spec = pl.BlockSpec((block_rows, features), lambda i: (i, 0))

    out = pl.pallas_call(
        functools.partial(_fused_ln_ff_residual_kernel, eps=eps),
        out_shape=jax.ShapeDtypeStruct((rows, features), x.dtype),
        grid=grid,
        in_specs=[
            row_spec,
            pl.BlockSpec((1, features), lambda i: (0, 0)),
            pl.BlockSpec((1, features), lambda i: (0, 0)),
            pl.BlockSpec((features, features), lambda i: (0, 0)),
        ],
        out_specs=row_spec,
        compiler_params=pltpu.CompilerParams(
            dimension_semantics=("parallel",),
            vmem_limit_bytes=48 * 1024 * 1024,
        ),
    )(x2, g2, b2, w_bf16)

    return out.reshape(orig_shape)
